# baseline (speedup 1.0000x reference)
import functools

import jax
import jax.numpy as jnp
from jax import lax
from jax.experimental import pallas as pl
from jax.experimental.pallas import tpu as pltpu
from jax.experimental.pallas import tpu_sc as plsc

L = 16
N = 256
NCHUNK = N // L
NC = 2
B = 16
BIG = 1 << 30


def _argmax16(load_chunk, lane_off=None):
    lane = lax.iota(jnp.int32, L)
    if lane_off is None:
        lane_off = lane
    pairs = [(load_chunk(j), jnp.full((L,), jnp.int32(j * L)))
             for j in range(NCHUNK)]
    while len(pairs) > 1:
        nxt = []
        for k in range(0, len(pairs), 2):
            (va, ia), (vb, ib) = pairs[k], pairs[k + 1]
            take_b = vb > va
            nxt.append((jnp.where(take_b, vb, va),
                        jnp.where(take_b, ib, ia)))
        pairs = nxt
    vmax, vbase = pairs[0]
    m = jnp.max(vmax)
    cand = jnp.where(vmax == m, vbase + lane_off, BIG)
    idx = jnp.min(cand)
    return m, idx


def _make_kernel():
    mesh = plsc.VectorSubcoreMesh(core_axis_name="c", subcore_axis_name="s")

    @functools.partial(
        pl.kernel,
        mesh=mesh,
        out_type=jax.ShapeDtypeStruct((B, N, N), jnp.float32),
        compiler_params=pltpu.CompilerParams(needs_layout_passes=False),
        scratch_types=[
            pltpu.VMEM((N, N), jnp.float32),
            pltpu.VMEM((N,), jnp.float32),
            pltpu.VMEM((N,), jnp.int32),
            pltpu.VMEM((N,), jnp.int32),
        ],
    )
    def greedy(s_hbm, out_hbm, st, cached, picks_r, picks_c):
        wid = lax.axis_index("s") * NC + lax.axis_index("c")

        @pl.when(wid < B)
        def _body():
            lane = lax.iota(jnp.int32, L)
            lane0 = lane == 0
            zero = jnp.zeros((L,), jnp.float32)
            ones = jnp.ones((L,), jnp.float32)

            pltpu.sync_copy(s_hbm.at[wid], st)

            def set1(ref, i, val):
                plsc.store_scatter(ref, [jnp.broadcast_to(i, (L,))],
                                   jnp.broadcast_to(val, (L,)), mask=lane0)

            def init_row(rr, carry):
                rs = jnp.broadcast_to(rr, (L,))
                perm_c = (lane + rs) & 15
                vmax = None
                for j in range(NCHUNK):
                    v = st[rr, pl.ds(j * L, L)]
                    plsc.store_scatter(st, [rs, perm_c + jnp.int32(j * L)],
                                       v)
                    vmax = v if j == 0 else jnp.maximum(vmax, v)
                set1(cached, rr, jnp.max(vmax))
                return carry
            lax.fori_loop(0, N, init_row, 0)

            def setv(vt, i, val):
                return tuple(
                    jnp.where(lane == jnp.broadcast_to(i - k * L, (L,)),
                              val, vt[k])
                    for k in range(NCHUNK))

            def pick(ca):
                def rescan(rr):
                    orig_lane = (lane - jnp.broadcast_to(rr, (L,))) & 15
                    return _argmax16(
                        lambda j: st[rr, pl.ds(j * L, L)], orig_lane)

                m, r = _argmax16(lambda j: ca[j])
                tm, c = rescan(r)

                def stale(cy):
                    return cy[0] != cy[1]

                def fix(cy):
                    m_, tm_, r_, _, ca_ = cy
                    ca2 = setv(ca_, r_, tm_)
                    m2, r2 = _argmax16(lambda j: ca2[j])
                    tm2, c2 = rescan(r2)
                    return (m2, tm2, r2, c2, ca2)

                m, tm, r, c, ca = lax.while_loop(
                    stale, fix, (m, tm, r, c, ca))
                return m, r, c, ca

            def commit(i, r, c, ca):
                for j in range(NCHUNK):
                    st[r, pl.ds(j * L, L)] = zero
                cs = jnp.broadcast_to(c, (L,))
                colpos = jnp.broadcast_to(c & (-16), (L,)) \
                    + ((cs + lane) & 15)
                for j in range(NCHUNK):
                    plsc.store_scatter(
                        st, [lane + jnp.int32(j * L), colpos], zero)
                set1(picks_r, i, r)
                set1(picks_c, i, c)
                return setv(ca, r, jnp.float32(0.0))

            ca0 = tuple(cached[pl.ds(j * L, L)] for j in range(NCHUNK))
            m0, r0, c0, ca0 = pick(ca0)

            def step(i, carry):
                r_p, c_p, ca, dirty = carry
                ca = commit(i - 1, r_p, c_p, ca)
                m, r, c, ca = pick(ca)
                dirty = jnp.where(m == 0.0, jnp.int32(1), dirty)
                return (r, c, ca, dirty)

            r_l, c_l, ca_l, dirty = lax.fori_loop(
                1, N, step,
                (r0, c0, ca0,
                 jnp.where(m0 == 0.0, jnp.int32(1), jnp.int32(0))))
            commit(jnp.int32(N - 1), r_l, c_l, ca_l)

            @pl.when(dirty == 1)
            def _clear():
                def clear_row(rr, carry):
                    for j in range(NCHUNK):
                        st[rr, pl.ds(j * L, L)] = zero
                    return carry
                lax.fori_loop(0, N, clear_row, 0)

            for j in range(NCHUNK):
                plsc.store_scatter(
                    st, [picks_r[pl.ds(j * L, L)], picks_c[pl.ds(j * L, L)]],
                    ones)

            pltpu.sync_copy(st, out_hbm.at[wid])

    return greedy


_greedy_kernel = _make_kernel()


@jax.jit
def kernel(s):
    return _greedy_kernel(s)


# speedup vs baseline: 1.1588x; 1.1588x over previous
import functools

import jax
import jax.numpy as jnp
from jax import lax
from jax.experimental import pallas as pl
from jax.experimental.pallas import tpu as pltpu
from jax.experimental.pallas import tpu_sc as plsc

L = 16
N = 256
NCHUNK = N // L
NC = 2
B = 16
BIG = 1 << 30


def _argmax16(load_chunk):
    lane = lax.iota(jnp.int32, L)
    pairs = [(load_chunk(j), jnp.full((L,), jnp.int32(j * L)))
             for j in range(NCHUNK)]
    while len(pairs) > 1:
        nxt = []
        for k in range(0, len(pairs), 2):
            (va, ia), (vb, ib) = pairs[k], pairs[k + 1]
            take_b = vb > va
            nxt.append((jnp.where(take_b, vb, va),
                        jnp.where(take_b, ib, ia)))
        pairs = nxt
    vmax, vbase = pairs[0]
    m = jnp.max(vmax)
    cand = jnp.where(vmax == m, vbase + lane, BIG)
    idx = jnp.min(cand)
    return m, idx


def _make_kernel():
    mesh = plsc.VectorSubcoreMesh(core_axis_name="c", subcore_axis_name="s")

    @functools.partial(
        pl.kernel,
        mesh=mesh,
        out_type=jax.ShapeDtypeStruct((B, N, N), jnp.float32),
        compiler_params=pltpu.CompilerParams(needs_layout_passes=False),
        scratch_types=[
            pltpu.VMEM((N, N), jnp.float32),
            pltpu.VMEM((N,), jnp.float32),
            pltpu.VMEM((N,), jnp.int32),
            pltpu.VMEM((N,), jnp.int32),
        ],
    )
    def greedy(s_hbm, out_hbm, st, cached, picks_r, picks_c):
        wid = lax.axis_index("s") * NC + lax.axis_index("c")

        @pl.when(wid < B)
        def _body():
            lane = lax.iota(jnp.int32, L)
            lane0 = lane == 0
            zero = jnp.zeros((L,), jnp.float32)
            ones = jnp.ones((L,), jnp.float32)

            pltpu.sync_copy(s_hbm.at[wid], st)

            def set1(ref, i, val):
                plsc.store_scatter(ref, [jnp.broadcast_to(i, (L,))],
                                   jnp.broadcast_to(val, (L,)), mask=lane0)

            def init_row(rr, carry):
                vmax = st[rr, pl.ds(0, L)]
                for j in range(1, NCHUNK):
                    vmax = jnp.maximum(vmax, st[rr, pl.ds(j * L, L)])
                set1(cached, rr, jnp.max(vmax))
                return carry
            lax.fori_loop(0, N, init_row, 0)

            def setv(vt, i, val):
                return tuple(
                    jnp.where(lane == jnp.broadcast_to(i - k * L, (L,)),
                              val, vt[k])
                    for k in range(NCHUNK))

            def pick(ca, cm):
                def rescan(rr):
                    return _argmax16(
                        lambda j: st[rr, pl.ds(j * L, L)] * cm[j])

                m, r = _argmax16(lambda j: ca[j])
                tm, c = rescan(r)

                def stale(cy):
                    return cy[0] != cy[1]

                def fix(cy):
                    m_, tm_, r_, _, ca_ = cy
                    ca2 = setv(ca_, r_, tm_)
                    m2, r2 = _argmax16(lambda j: ca2[j])
                    tm2, c2 = rescan(r2)
                    return (m2, tm2, r2, c2, ca2)

                m, tm, r, c, ca = lax.while_loop(
                    stale, fix, (m, tm, r, c, ca))
                return m, r, c, ca

            def commit(i, r, c, ca, cm):
                for j in range(NCHUNK):
                    st[r, pl.ds(j * L, L)] = zero
                set1(picks_r, i, r)
                set1(picks_c, i, c)
                return setv(ca, r, jnp.float32(0.0)), \
                    setv(cm, c, jnp.float32(0.0))

            ca0 = tuple(cached[pl.ds(j * L, L)] for j in range(NCHUNK))
            cm0 = tuple(ones for _ in range(NCHUNK))
            m0, r0, c0, ca0 = pick(ca0, cm0)

            def step(i, carry):
                r_p, c_p, ca, cm, dirty = carry
                ca, cm = commit(i - 1, r_p, c_p, ca, cm)
                m, r, c, ca = pick(ca, cm)
                dirty = jnp.where(m == 0.0, jnp.int32(1), dirty)
                return (r, c, ca, cm, dirty)

            r_l, c_l, ca_l, cm_l, dirty = lax.fori_loop(
                1, N, step,
                (r0, c0, ca0, cm0,
                 jnp.where(m0 == 0.0, jnp.int32(1), jnp.int32(0))))
            commit(jnp.int32(N - 1), r_l, c_l, ca_l, cm_l)

            @pl.when(dirty == 1)
            def _clear():
                def clear_row(rr, carry):
                    for j in range(NCHUNK):
                        st[rr, pl.ds(j * L, L)] = zero
                    return carry
                lax.fori_loop(0, N, clear_row, 0)

            for j in range(NCHUNK):
                plsc.store_scatter(
                    st, [picks_r[pl.ds(j * L, L)], picks_c[pl.ds(j * L, L)]],
                    ones)

            pltpu.sync_copy(st, out_hbm.at[wid])

    return greedy


_greedy_kernel = _make_kernel()


@jax.jit
def kernel(s):
    return _greedy_kernel(s)
